# baseline (speedup 1.0000x reference)
import jax
import jax.numpy as jnp
from jax import lax
from jax.experimental import pallas as pl
from jax.experimental.pallas import tpu as pltpu
from jax.experimental.pallas import tpu_sc as plsc

N = 10000
E = 320000
D = 128
NC = 2
NS = 16
NW = NC * NS
C = 128
EP = 327680
EPW = EP // NW
CHUNKS = EPW // C
HCH = CHUNKS // 2
NP = 10240
RPT = NP // NS
ZROWS = 128

_F32 = jnp.float32
_HIGHEST = lax.Precision.HIGHEST


def _matmul_t(a, w):
    return lax.dot_general(a, w, (((1,), (1,)), ((), ())),
                           precision=_HIGHEST, preferred_element_type=_F32)


def _ln(t, g, b):
    mu = jnp.mean(t, axis=1, keepdims=True)
    d = t - mu
    var = jnp.mean(d * d, axis=1, keepdims=True)
    return d * lax.rsqrt(var + 1e-5) * g + b


G = 4
SG = C // G


def _sc_agg_body(y_h, src_h, dst_h, za_h, psum_h,
                 src_v, dst_v, rows0, rows1, acc, *sems):
    cid = lax.axis_index("c")
    sid = lax.axis_index("s")
    wid = sid * NC + cid
    bufs = (rows0, rows1)

    for k in range(RPT // ZROWS):
        pltpu.sync_copy(za_h, acc.at[pl.ds(sid * RPT + k * ZROWS, ZROWS)])
    plsc.subcore_barrier()

    def start(c, b):
        for q in range(G):
            pltpu.async_copy(y_h.at[src_v.at[c, pl.ds(q * SG, SG)]],
                             bufs[b].at[pl.ds(q * SG, SG)], sems[b * G + q])

    def wait(c, b):
        for q in range(G):
            pltpu.make_async_copy(y_h.at[src_v.at[c, pl.ds(q * SG, SG)]],
                                  bufs[b].at[pl.ds(q * SG, SG)],
                                  sems[b * G + q]).wait()

    for p in range(2):
        pltpu.sync_copy(src_h.at[wid, pl.ds(p * HCH, HCH)], src_v)
        pltpu.sync_copy(dst_h.at[wid, pl.ds(p * HCH, HCH)], dst_v)

        start(0, 0)

        def mbody(g, carry):
            c0 = 2 * g
            wait(c0, 0)
            start(c0 + 1, 1)
            pltpu.sync_copy(rows0, acc.at[dst_v.at[c0]], add=True)
            wait(c0 + 1, 1)

            @pl.when(g < HCH // 2 - 1)
            def _():
                start(c0 + 2, 0)

            pltpu.sync_copy(rows1, acc.at[dst_v.at[c0 + 1]], add=True)
            return carry

        lax.fori_loop(0, HCH // 2, mbody, 0)

    plsc.subcore_barrier()

    for k in range(RPT // ZROWS):
        r0 = sid * RPT + k * ZROWS
        pltpu.sync_copy(acc.at[pl.ds(r0, ZROWS)], psum_h.at[cid, pl.ds(r0, ZROWS)])


def _make_sc_agg():
    mesh = plsc.VectorSubcoreMesh(core_axis_name="c", subcore_axis_name="s")
    return pl.kernel(
        _sc_agg_body,
        out_type=[jax.ShapeDtypeStruct((NC, NP, D), _F32)],
        mesh=mesh,
        scratch_types=[
            pltpu.VMEM((HCH, C), jnp.int32),
            pltpu.VMEM((HCH, C), jnp.int32),
            pltpu.VMEM((C, D), _F32),
            pltpu.VMEM((C, D), _F32),
            pltpu.VMEM_SHARED((NP, D), _F32),
        ] + [pltpu.SemaphoreType.DMA] * (2 * G),
    )


_sc_agg_1 = _make_sc_agg()
_sc_agg_2 = _make_sc_agg()


def _sc_cnt_body(dst_h, za_h, ones_h, cnt_h, dst_v, ones_v, cnt_sh):
    cid = lax.axis_index("c")
    sid = lax.axis_index("s")
    wid = sid * NC + cid

    for k in range(RPT // ZROWS):
        pltpu.sync_copy(za_h, cnt_sh.at[pl.ds(sid * RPT + k * ZROWS, ZROWS)])
    pltpu.sync_copy(ones_h, ones_v)
    pltpu.sync_copy(dst_h.at[wid], dst_v)
    plsc.subcore_barrier()

    def cbody(j, carry):
        pltpu.sync_copy(ones_v, cnt_sh.at[dst_v.at[j]], add=True)
        return carry

    lax.fori_loop(0, CHUNKS, cbody, 0)

    plsc.subcore_barrier()
    for k in range(RPT // ZROWS):
        r0 = sid * RPT + k * ZROWS
        pltpu.sync_copy(cnt_sh.at[pl.ds(r0, ZROWS)], cnt_h.at[cid, pl.ds(r0, ZROWS)])


_sc_cnt = pl.kernel(
    _sc_cnt_body,
    out_type=[jax.ShapeDtypeStruct((NC, NP, D), _F32)],
    mesh=plsc.VectorSubcoreMesh(core_axis_name="c", subcore_axis_name="s"),
    scratch_types=[
        pltpu.VMEM((CHUNKS, C), jnp.int32),
        pltpu.VMEM((C, D), _F32),
        pltpu.VMEM_SHARED((NP, D), _F32),
    ],
)


def _tc1_body(x_ref, wl_ref, bl_ref, wr_ref, y_ref, z_ref):
    x = x_ref[...]
    y_ref[...] = _matmul_t(x, wl_ref[...])
    z_ref[...] = _matmul_t(x, wr_ref[...]) + bl_ref[...]


def _tc2_body(p_ref, cnt_ref, z_ref, g_ref, b_ref, wl_ref, bl_ref, wr_ref,
              y_ref, z2_ref):
    inv = 1.0 / jnp.maximum(cnt_ref[0] + cnt_ref[1], 1.0)
    t = (p_ref[0, pl.ds(0, N)] + p_ref[1, pl.ds(0, N)]) * inv + z_ref[...]
    h = jnp.maximum(_ln(t, g_ref[...], b_ref[...]), 0.0)
    y_ref[...] = _matmul_t(h, wl_ref[...])
    z2_ref[...] = _matmul_t(h, wr_ref[...]) + bl_ref[...]


def _tc3_body(p_ref, cnt_ref, z_ref, g_ref, b_ref, wr1_ref, br1_ref, gr_ref,
              ber_ref, wr2_ref, br2_ref, o_ref):
    inv = 1.0 / jnp.maximum(cnt_ref[0] + cnt_ref[1], 1.0)
    t = (p_ref[0, pl.ds(0, N)] + p_ref[1, pl.ds(0, N)]) * inv + z_ref[...]
    h = jnp.maximum(_ln(t, g_ref[...], b_ref[...]), 0.0)
    r = _matmul_t(h, wr1_ref[...]) + br1_ref[...]
    r = jnp.maximum(_ln(r, gr_ref[...], ber_ref[...]), 0.0)
    o_ref[...] = _matmul_t(r, wr2_ref[...]) + br2_ref[...]


_tc1 = pl.pallas_call(
    _tc1_body,
    out_shape=[jax.ShapeDtypeStruct((N, D), _F32),
               jax.ShapeDtypeStruct((N, D), _F32)],
)

_tc2 = pl.pallas_call(
    _tc2_body,
    out_shape=[jax.ShapeDtypeStruct((N, D), _F32),
               jax.ShapeDtypeStruct((N, D), _F32)],
)

_tc3 = pl.pallas_call(
    _tc3_body,
    out_shape=jax.ShapeDtypeStruct((N, 64), _F32),
)


def kernel(x, edge_index, edge_weight, W1l, b1l, W1r, g1, be1,
           W2l, b2l, W2r, g2, be2, Wr1, br1, gr, ber, Wr2, br2):
    del edge_weight
    src = edge_index[0].astype(jnp.int32)
    dst = edge_index[1].astype(jnp.int32)
    pad = jnp.zeros((EP - E,), jnp.int32)
    src2 = jnp.concatenate([src, pad]).reshape(NW, CHUNKS, C)
    dst2 = jnp.concatenate([dst, pad + N]).reshape(NW, CHUNKS, C)
    zeros_a = jnp.zeros((ZROWS, D), _F32)
    ones_rows = jnp.zeros((C, D), _F32).at[:, 0].set(1.0)

    cnt = _sc_cnt(dst2, zeros_a, ones_rows)[0]
    cnt_col = cnt[:, :N, 0:1]
    y1, z1 = _tc1(x, W1l, b1l[None], W1r)
    psum1 = _sc_agg_1(y1, src2, dst2, zeros_a)[0]
    y2, z2 = _tc2(psum1, cnt_col, z1, g1[None], be1[None], W2l, b2l[None], W2r)
    psum2 = _sc_agg_2(y2, src2, dst2, zeros_a)[0]
    out = _tc3(psum2, cnt_col, z2, g2[None], be2[None], Wr1, br1[None],
               gr[None], ber[None], Wr2, br2[None])
    return out


# speedup vs baseline: 1.5953x; 1.5953x over previous
import jax
import jax.numpy as jnp
from jax import lax
from jax.experimental import pallas as pl
from jax.experimental.pallas import tpu as pltpu
from jax.experimental.pallas import tpu_sc as plsc

N = 10000
E = 320000
D = 128
NC = 2
NS = 16
NW = NC * NS
C = 32
EP = 327680
EPW = EP // NW
CHUNKS = EPW // C
NP = 10240
RPT = NP // NS
ZROWS = 128
HN = 5120
ACCR = 5128
DUMP = 5120
ZACC = 328
YRT = 624

_F32 = jnp.float32
_HIGHEST = lax.Precision.HIGHEST


def _matmul_t(a, w):
    return lax.dot_general(a, w, (((1,), (1,)), ((), ())),
                           precision=_HIGHEST, preferred_element_type=_F32)


def _ln(t, g, b):
    mu = jnp.mean(t, axis=1, keepdims=True)
    d = t - mu
    var = jnp.mean(d * d, axis=1, keepdims=True)
    return d * lax.rsqrt(var + 1e-5) * g + b


def _sc_agg_body(y_h, src_h, dl0_h, dl1_h, za_h, psum_h, *refs):
    sb = refs[0:4]
    db = refs[4:8]
    rows = refs[8:10]
    ytab, acc = refs[10], refs[11]
    semis = refs[12:16]
    semid = refs[16:20]
    semr = refs[20:22]
    cid = lax.axis_index("c")
    sid = lax.axis_index("s")
    wid = sid * NC + cid
    base = wid * EPW

    pltpu.sync_copy(y_h.at[pl.ds(sid * YRT, YRT)], ytab.at[pl.ds(sid * YRT, YRT)])

    @pl.when(sid == 0)
    def _():
        pltpu.sync_copy(y_h.at[pl.ds(NS * YRT, N - NS * YRT)],
                        ytab.at[pl.ds(NS * YRT, N - NS * YRT)])

    for h, dl_h in enumerate((dl0_h, dl1_h)):

        def idx_start(c, slot):
            pltpu.async_copy(src_h.at[pl.ds(base + c * C, C)], sb[slot], semis[slot])
            pltpu.async_copy(dl_h.at[pl.ds(base + c * C, C)], db[slot], semid[slot])

        def idx_wait(c, slot):
            pltpu.make_async_copy(src_h.at[pl.ds(base + c * C, C)], sb[slot],
                                  semis[slot]).wait()
            pltpu.make_async_copy(dl_h.at[pl.ds(base + c * C, C)], db[slot],
                                  semid[slot]).wait()

        def gather_start(slot, b):
            pltpu.async_copy(ytab.at[sb[slot]], rows[b], semr[b])

        def gather_wait(slot, b):
            pltpu.make_async_copy(ytab.at[sb[slot]], rows[b], semr[b]).wait()

        @pl.when(sid < NS - 1)
        def _():
            pltpu.sync_copy(za_h.at[pl.ds(0, ZACC - 8)], acc.at[pl.ds(sid * (ZACC - 8), ZACC - 8)])

        @pl.when(sid == NS - 1)
        def _():
            pltpu.sync_copy(za_h, acc.at[pl.ds((NS - 1) * (ZACC - 8), ZACC)])

        plsc.subcore_barrier()

        for c in range(3):
            idx_start(c, c)
        idx_wait(0, 0)
        gather_start(0, 0)

        def mbody(g, carry):
            for k in range(4):
                c = 4 * g + k

                @pl.when(c + 3 < CHUNKS)
                def _():
                    idx_start(c + 3, (k + 3) % 4)

                @pl.when(c + 1 < CHUNKS)
                def _():
                    idx_wait(c + 1, (k + 1) % 4)
                    gather_start((k + 1) % 4, (k + 1) % 2)

                gather_wait(k % 4, k % 2)
                pltpu.sync_copy(rows[k % 2], acc.at[db[k % 4]], add=True)
            return carry

        lax.fori_loop(0, CHUNKS // 4, mbody, 0)
        plsc.subcore_barrier()

        pltpu.sync_copy(acc.at[pl.ds(sid * (HN // NS), HN // NS)],
                        psum_h.at[cid, pl.ds(h * HN + sid * (HN // NS), HN // NS)])
        plsc.subcore_barrier()


def _make_sc_agg():
    mesh = plsc.VectorSubcoreMesh(core_axis_name="c", subcore_axis_name="s")
    return pl.kernel(
        _sc_agg_body,
        out_type=[jax.ShapeDtypeStruct((NC, NP, D), _F32)],
        mesh=mesh,
        scratch_types=(
            [pltpu.VMEM((C,), jnp.int32)] * 4
            + [pltpu.VMEM((C,), jnp.int32)] * 4
            + [pltpu.VMEM((C, D), _F32)] * 2
            + [pltpu.VMEM_SHARED((N, D), _F32),
               pltpu.VMEM_SHARED((ACCR, D), _F32)]
            + [pltpu.SemaphoreType.DMA] * 10
        ),
    )


_sc_agg_1 = _make_sc_agg()
_sc_agg_2 = _make_sc_agg()


def _sc_cnt_body(dst_h, za_h, ones_h, cnt_h, dst_v, ones_v, cnt_sh):
    cid = lax.axis_index("c")
    sid = lax.axis_index("s")
    wid = sid * NC + cid

    for k in range(RPT // ZROWS):
        pltpu.sync_copy(za_h, cnt_sh.at[pl.ds(sid * RPT + k * ZROWS, ZROWS)])
    pltpu.sync_copy(ones_h, ones_v)
    pltpu.sync_copy(dst_h.at[wid], dst_v)
    plsc.subcore_barrier()

    def cbody(j, carry):
        pltpu.sync_copy(ones_v, cnt_sh.at[dst_v.at[j]], add=True)
        return carry

    lax.fori_loop(0, EPW // 128, cbody, 0)

    plsc.subcore_barrier()
    for k in range(RPT // ZROWS):
        r0 = sid * RPT + k * ZROWS
        pltpu.sync_copy(cnt_sh.at[pl.ds(r0, ZROWS)], cnt_h.at[cid, pl.ds(r0, ZROWS)])


_sc_cnt = pl.kernel(
    _sc_cnt_body,
    out_type=[jax.ShapeDtypeStruct((NC, NP, D), _F32)],
    mesh=plsc.VectorSubcoreMesh(core_axis_name="c", subcore_axis_name="s"),
    scratch_types=[
        pltpu.VMEM((EPW // 128, 128), jnp.int32),
        pltpu.VMEM((128, D), _F32),
        pltpu.VMEM_SHARED((NP, D), _F32),
    ],
)


def _tc1_body(x_ref, wl_ref, bl_ref, wr_ref, y_ref, z_ref):
    x = x_ref[...]
    y_ref[...] = _matmul_t(x, wl_ref[...])
    z_ref[...] = _matmul_t(x, wr_ref[...]) + bl_ref[...]


def _tc2_body(p_ref, cnt_ref, z_ref, g_ref, b_ref, wl_ref, bl_ref, wr_ref,
              y_ref, z2_ref):
    inv = 1.0 / jnp.maximum(cnt_ref[0] + cnt_ref[1], 1.0)
    t = (p_ref[0, pl.ds(0, N)] + p_ref[1, pl.ds(0, N)]) * inv + z_ref[...]
    h = jnp.maximum(_ln(t, g_ref[...], b_ref[...]), 0.0)
    y_ref[...] = _matmul_t(h, wl_ref[...])
    z2_ref[...] = _matmul_t(h, wr_ref[...]) + bl_ref[...]


def _tc3_body(p_ref, cnt_ref, z_ref, g_ref, b_ref, wr1_ref, br1_ref, gr_ref,
              ber_ref, wr2_ref, br2_ref, o_ref):
    inv = 1.0 / jnp.maximum(cnt_ref[0] + cnt_ref[1], 1.0)
    t = (p_ref[0, pl.ds(0, N)] + p_ref[1, pl.ds(0, N)]) * inv + z_ref[...]
    h = jnp.maximum(_ln(t, g_ref[...], b_ref[...]), 0.0)
    r = _matmul_t(h, wr1_ref[...]) + br1_ref[...]
    r = jnp.maximum(_ln(r, gr_ref[...], ber_ref[...]), 0.0)
    o_ref[...] = _matmul_t(r, wr2_ref[...]) + br2_ref[...]


_tc1 = pl.pallas_call(
    _tc1_body,
    out_shape=[jax.ShapeDtypeStruct((N, D), _F32),
               jax.ShapeDtypeStruct((N, D), _F32)],
)

_tc2 = pl.pallas_call(
    _tc2_body,
    out_shape=[jax.ShapeDtypeStruct((N, D), _F32),
               jax.ShapeDtypeStruct((N, D), _F32)],
)

_tc3 = pl.pallas_call(
    _tc3_body,
    out_shape=jax.ShapeDtypeStruct((N, 64), _F32),
)


def kernel(x, edge_index, edge_weight, W1l, b1l, W1r, g1, be1,
           W2l, b2l, W2r, g2, be2, Wr1, br1, gr, ber, Wr2, br2):
    del edge_weight
    src = edge_index[0].astype(jnp.int32)
    dst = edge_index[1].astype(jnp.int32)
    pad = jnp.zeros((EP - E,), jnp.int32)
    srcp = jnp.concatenate([src, pad])
    dstp = jnp.concatenate([dst, pad + N])
    dl0 = jnp.where(dstp < HN, dstp, DUMP)
    dl1 = jnp.where(dstp >= HN, dstp - HN, DUMP)
    dst128 = dstp.reshape(NW, EPW // 128, 128)
    zeros_a = jnp.zeros((ZACC, D), _F32)
    zeros_z = jnp.zeros((ZROWS, D), _F32)
    ones_rows = jnp.zeros((128, D), _F32).at[:, 0].set(1.0)

    cnt = _sc_cnt(dst128, zeros_z, ones_rows)[0]
    cnt_col = cnt[:, :N, 0:1]
    y1, z1 = _tc1(x, W1l, b1l[None], W1r)
    psum1 = _sc_agg_1(y1, srcp, dl0, dl1, zeros_a)[0]
    y2, z2 = _tc2(psum1, cnt_col, z1, g1[None], be1[None], W2l, b2l[None], W2r)
    psum2 = _sc_agg_2(y2, srcp, dl0, dl1, zeros_a)[0]
    out = _tc3(psum2, cnt_col, z2, g2[None], be2[None], Wr1, br1[None],
               gr[None], ber[None], Wr2, br2[None])
    return out
